# baseline (speedup 1.0000x reference)
import functools

import jax
import jax.numpy as jnp
from jax import lax
from jax.experimental import pallas as pl
from jax.experimental.pallas import tpu as pltpu
from jax.experimental.pallas import tpu_sc as plsc

B = 16384
D = 32
WD = 1e-05
NC = 2
NS = 16
NW = NC * NS
BPW = B // NW
CHUNK = 128
NCHUNK = BPW // CHUNK


def _sc_body(u_hbm, i_hbm, j_hbm, w_hbm, h_hbm, out_hbm,
             idx_u, idx_i, idx_j, rows_u, rows_i, rows_j,
             x_v, uu_v, ii_v, jj_v, sem_idx, sem_rows):
    cid = lax.axis_index("c")
    sid = lax.axis_index("s")
    wid = sid * NC + cid
    base = wid * BPW

    idx_copies = []
    for k in range(NCHUNK):
        sl = pl.ds(base + k * CHUNK, CHUNK)
        idx_copies.append(pltpu.async_copy(u_hbm.at[sl], idx_u.at[k], sem_idx))
        idx_copies.append(pltpu.async_copy(i_hbm.at[sl], idx_i.at[k], sem_idx))
        idx_copies.append(pltpu.async_copy(j_hbm.at[sl], idx_j.at[k], sem_idx))
    for c in idx_copies:
        c.wait()

    row_copies = []
    for k in range(NCHUNK):
        dst = pl.ds(k * CHUNK, CHUNK)
        row_copies.append(
            pltpu.async_copy(w_hbm.at[idx_u.at[k]], rows_u.at[dst], sem_rows))
        row_copies.append(
            pltpu.async_copy(h_hbm.at[idx_i.at[k]], rows_i.at[dst], sem_rows))
        row_copies.append(
            pltpu.async_copy(h_hbm.at[idx_j.at[k]], rows_j.at[dst], sem_rows))
    for c in row_copies:
        c.wait()

    lane = lax.iota(jnp.int32, 16)

    def group(g, carry):
        row_ids = g * 16 + lane
        xa = jnp.zeros((16,), jnp.float32)
        ua = jnp.zeros((16,), jnp.float32)
        ia = jnp.zeros((16,), jnp.float32)
        ja = jnp.zeros((16,), jnp.float32)
        for d in range(D):
            dcol = jnp.full((16,), d, jnp.int32)
            cu = plsc.load_gather(rows_u, [row_ids, dcol])
            ci = plsc.load_gather(rows_i, [row_ids, dcol])
            cj = plsc.load_gather(rows_j, [row_ids, dcol])
            xa = xa + cu * (ci - cj)
            ua = ua + cu * cu
            ia = ia + ci * ci
            ja = ja + cj * cj
        sl = pl.ds(g * 16, 16)
        x_v[sl] = xa
        uu_v[sl] = ua
        ii_v[sl] = ia
        jj_v[sl] = ja
        return carry

    lax.fori_loop(0, BPW // 16, group, 0)

    pltpu.sync_copy(x_v, out_hbm.at[pl.ds(0 * B + base, BPW)])
    pltpu.sync_copy(uu_v, out_hbm.at[pl.ds(1 * B + base, BPW)])
    pltpu.sync_copy(ii_v, out_hbm.at[pl.ds(2 * B + base, BPW)])
    pltpu.sync_copy(jj_v, out_hbm.at[pl.ds(3 * B + base, BPW)])


_sc_call = functools.partial(
    pl.kernel,
    out_type=jax.ShapeDtypeStruct((4 * B,), jnp.float32),
    mesh=plsc.VectorSubcoreMesh(core_axis_name="c", subcore_axis_name="s"),
    compiler_params=pltpu.CompilerParams(
        needs_layout_passes=False, use_tc_tiling_on_sc=False),
    scratch_types=[
        pltpu.VMEM((NCHUNK, CHUNK), jnp.int32),
        pltpu.VMEM((NCHUNK, CHUNK), jnp.int32),
        pltpu.VMEM((NCHUNK, CHUNK), jnp.int32),
        pltpu.VMEM((BPW, D), jnp.float32),
        pltpu.VMEM((BPW, D), jnp.float32),
        pltpu.VMEM((BPW, D), jnp.float32),
        pltpu.VMEM((BPW,), jnp.float32),
        pltpu.VMEM((BPW,), jnp.float32),
        pltpu.VMEM((BPW,), jnp.float32),
        pltpu.VMEM((BPW,), jnp.float32),
        pltpu.SemaphoreType.DMA,
        pltpu.SemaphoreType.DMA,
    ],
)(_sc_body)


def _tc_body(o_ref, out_ref):
    x = o_ref[pl.ds(0, 128), :]
    uu = o_ref[pl.ds(128, 128), :]
    ii = o_ref[pl.ds(256, 128), :]
    jj = o_ref[pl.ds(384, 128), :]
    reg = WD * (jnp.sqrt(uu) + jnp.sqrt(ii) + jnp.sqrt(jj))
    out_ref[...] = -jax.nn.log_sigmoid(x) + reg


_tc_call = pl.pallas_call(
    _tc_body,
    out_shape=jax.ShapeDtypeStruct((128, 128), jnp.float32),
)


def kernel(u, i, j, W, H):
    u = u.astype(jnp.int32)
    i = i.astype(jnp.int32)
    j = j.astype(jnp.int32)
    packed = _sc_call(u, i, j, W, H)
    return _tc_call(packed.reshape(512, 128)).reshape(B)
